# Initial kernel scaffold; baseline (speedup 1.0000x reference)
import jax
import jax.numpy as jnp
from jax.experimental import pallas as pl


def kernel(x):
    raise NotImplementedError("write your pallas kernel here")


# speedup vs baseline: 1.2175x; 1.2175x over previous
import jax
import jax.numpy as jnp
from jax.experimental import pallas as pl

_K = 16


def _row_topk_sum_kernel(x_ref, o_ref):
    nchunks = x_ref.shape[1] // 8
    neg = jnp.float32(-jnp.inf)

    init = [jnp.full((8, 128), neg, dtype=jnp.float32) for _ in range(_K)]

    def insert_body(i, state):
        c = x_ref[0, pl.ds(i * 8, 8), :]
        new_state = []
        for k in range(_K):
            s = state[k]
            hi = jnp.maximum(s, c)
            c = jnp.minimum(s, c)
            new_state.append(hi)
        return new_state

    state = jax.lax.fori_loop(0, nchunks, insert_body, init)

    def merge_body(i, carry):
        acc_s, acc_n, tau = carry
        m = neg
        for k in range(_K):
            m = jnp.maximum(m, jnp.max(jnp.where(state[k] < tau, state[k], neg)))
        cnt = jnp.float32(0.0)
        for k in range(_K):
            cnt += jnp.sum((state[k] == m).astype(jnp.float32))
        take = jnp.minimum(cnt, jnp.float32(_K) - acc_n)
        contrib = jnp.where(take > 0.0, m, jnp.float32(0.0)) * take
        return acc_s + contrib, jnp.minimum(acc_n + cnt, jnp.float32(_K)), m

    acc_s, _, _ = jax.lax.fori_loop(
        0, _K, merge_body, (jnp.float32(0.0), jnp.float32(0.0), jnp.float32(jnp.inf))
    )
    o_ref[0] = jnp.full((8, 128), acc_s, dtype=jnp.float32)


def kernel(x):
    b, c, h, w = x.shape
    n = b * c
    hw = h * w
    assert hw % 1024 == 0, "row length must be a multiple of 8*128"
    rows = hw // 128
    xr = x.reshape(n, rows, 128)

    out = pl.pallas_call(
        _row_topk_sum_kernel,
        grid=(n,),
        in_specs=[pl.BlockSpec((1, rows, 128), lambda i: (i, 0, 0))],
        out_specs=pl.BlockSpec((1, 8, 128), lambda i: (i, 0, 0)),
        out_shape=jax.ShapeDtypeStruct((n, 8, 128), jnp.float32),
    )(xr)
    return out[:, 0, 0].reshape(b, c)


# speedup vs baseline: 5.7428x; 4.7171x over previous
import jax
import jax.numpy as jnp
from jax.experimental import pallas as pl

_K = 16
_GRP = 16


def _oddeven_sort_pairs(n):
    pairs = []

    def merge(lo, m, r):
        step = r * 2
        if step < m:
            merge(lo, m, step)
            merge(lo + r, m, step)
            for i in range(lo + r, lo + m - r, step):
                pairs.append((i, i + r))
        else:
            pairs.append((lo, lo + r))

    def sortnet(lo, m):
        if m > 1:
            h = m // 2
            sortnet(lo, h)
            sortnet(lo + h, h)
            merge(lo, m, 1)

    sortnet(0, n)
    return pairs


_SORT_PAIRS = _oddeven_sort_pairs(_GRP)


def _cmpx(lst, i, j):
    a, b = lst[i], lst[j]
    lst[i] = jnp.maximum(a, b)
    lst[j] = jnp.minimum(a, b)


def _merge_keep_topk(state, other):
    v = [jnp.maximum(state[k], other[_K - 1 - k]) for k in range(_K)]
    for d in (8, 4, 2, 1):
        for i in range(_K):
            if not i & d:
                _cmpx(v, i, i + d)
    return v


def _row_topk_sum_kernel(x_ref, o_ref):
    nchunks = x_ref.shape[1] // 8
    ngroups = nchunks // _GRP
    neg = jnp.float32(-jnp.inf)

    init = [jnp.full((8, 128), neg, dtype=jnp.float32) for _ in range(_K)]

    def insert_body(g, state):
        ch = [x_ref[0, pl.ds((g * _GRP + j) * 8, 8), :] for j in range(_GRP)]
        for (i, j) in _SORT_PAIRS:
            _cmpx(ch, i, j)
        return _merge_keep_topk(state, ch)

    state = jax.lax.fori_loop(0, ngroups, insert_body, init)

    folds = [(1, 64), (1, 32), (1, 16), (1, 8), (1, 4), (1, 2), (1, 1),
             (0, 4), (0, 2), (0, 1)]
    for axis, shift in folds[:-1]:
        rolled = [jnp.roll(s, shift, axis=axis) for s in state]
        state = _merge_keep_topk(state, rolled)
    axis, shift = folds[-1]
    rolled = [jnp.roll(s, shift, axis=axis) for s in state]
    total = jnp.zeros((8, 128), dtype=jnp.float32)
    for k in range(_K):
        total = total + jnp.maximum(state[k], rolled[_K - 1 - k])
    o_ref[0] = total


def kernel(x):
    b, c, h, w = x.shape
    n = b * c
    hw = h * w
    assert hw % (1024 * _GRP) == 0, "row length must be a multiple of 8*128*16"
    rows = hw // 128
    xr = x.reshape(n, rows, 128)

    out = pl.pallas_call(
        _row_topk_sum_kernel,
        grid=(n,),
        in_specs=[pl.BlockSpec((1, rows, 128), lambda i: (i, 0, 0))],
        out_specs=pl.BlockSpec((1, 8, 128), lambda i: (i, 0, 0)),
        out_shape=jax.ShapeDtypeStruct((n, 8, 128), jnp.float32),
    )(xr)
    return out[:, 0, 0].reshape(b, c)
